# baseline (speedup 1.0000x reference)
import functools

import jax
import jax.numpy as jnp
from jax import lax
from jax.experimental import pallas as pl
from jax.experimental.pallas import tpu as pltpu
from jax.experimental.pallas import tpu_sc as plsc

_BATCH = 16384
_HIST = 50
_D = 64

_info = plsc.get_sparse_core_info()
_NC, _NS = _info.num_cores, _info.num_subcores
_NW = _NC * _NS
_BPW = _BATCH // _NW

_NBUF = 8
_LEAD = 4


def _embed_gather(table, idx):
    mesh = plsc.VectorSubcoreMesh(core_axis_name="c", subcore_axis_name="s")

    @functools.partial(
        pl.kernel,
        out_type=jax.ShapeDtypeStruct((_BATCH, _HIST, _D), jnp.float32),
        mesh=mesh,
        scratch_types=(
            [
                pltpu.VMEM((_BPW, _HIST), jnp.int32),
                pltpu.VMEM((_NBUF, _HIST, _D), jnp.float32),
            ]
            + [pltpu.SemaphoreType.DMA] * _NBUF
            + [pltpu.SemaphoreType.DMA] * _NBUF
        ),
        compiler_params=pltpu.CompilerParams(use_tc_tiling_on_sc=False),
    )
    def k(table_hbm, idx_hbm, out_hbm, idx_v, rows_v, *sems):
        gsem = sems[:_NBUF]
        ssem = sems[_NBUF:]
        wid = lax.axis_index("s") * _NC + lax.axis_index("c")
        b0 = wid * _BPW

        pltpu.sync_copy(idx_hbm.at[pl.ds(b0, _BPW)], idx_v)

        def gather(r, b):
            pltpu.async_copy(table_hbm.at[idx_v.at[r]], rows_v.at[b], gsem[b])

        def gather_wait(b):
            pltpu.make_async_copy(
                table_hbm.at[idx_v.at[0]], rows_v.at[b], gsem[b]
            ).wait()

        def scatter(r, b):
            pltpu.async_copy(rows_v.at[b], out_hbm.at[b0 + r], ssem[b])

        def scatter_wait(b):
            pltpu.make_async_copy(
                rows_v.at[b], out_hbm.at[b0], ssem[b]
            ).wait()

        for b in range(_LEAD):
            gather(b, b)

        @pl.loop(0, _BPW, step=_NBUF)
        def _(s):
            for b in range(_NBUF):
                r = s + b
                gather_wait(b)
                scatter(r, b)
                nb = (b + _LEAD) % _NBUF
                nr = r + _LEAD

                @pl.when(nr < _BPW)
                def _():
                    @pl.when(nr >= _NBUF)
                    def _():
                        scatter_wait(nb)

                    gather(nr, nb)

        for b in range(_NBUF):
            scatter_wait(b)

    return k(table, idx)


def kernel(x, table):
    return _embed_gather(table, x.astype(jnp.int32))
